# baseline (speedup 1.0000x reference)
import functools

import jax
import jax.numpy as jnp
from jax.experimental import pallas as pl
from jax.experimental.pallas import tpu as pltpu

_TAU = 0.5
_B = 2048
_CH = 1024


def _body(bi_ref, bj_ref, w_ref, outr_ref, outc_ref, *, num_pairs):
    t = pl.program_id(0)
    bi = bi_ref[t]
    bj = bj_ref[t]

    @pl.when(t == 0)
    def _init():
        outr_ref[...] = jnp.zeros_like(outr_ref)
        outc_ref[...] = jnp.zeros_like(outc_ref)

    dn = (((1,), (1,)), ((), ()))
    wj = w_ref[pl.ds(bj * _B, _B), :]

    col8 = jnp.zeros((8, _B), jnp.float32)
    for r in range(_B // _CH):
        wi_r = w_ref[pl.ds(bi * _B + r * _CH, _CH), :]
        s_r = jax.lax.dot_general(wi_r, wj, dn, preferred_element_type=jnp.float32)
        e_r = jnp.exp2(s_r)
        rowp = e_r[:, 0:128]
        for g in range(1, _B // 128):
            rowp = rowp + e_r[:, g * 128 : (g + 1) * 128]
        outr_ref[pl.ds(bi, 1), pl.ds(r * _CH, _CH), :] += rowp[None]
        cpart = e_r[0:8, :]
        for k in range(1, _CH // 8):
            cpart = cpart + e_r[k * 8 : (k + 1) * 8, :]
        col8 = col8 + cpart

    @pl.when(bj != bi)
    def _col():
        outc_ref[pl.ds(bj, 1)] += col8[None, :, :]


@functools.partial(jax.jit, static_argnums=(1,))
def _rowsums(w, m):
    t_blocks = m // _B
    pairs = [(i, j) for i in range(t_blocks) for j in range(i, t_blocks)]
    num_pairs = len(pairs)
    bi_arr = jnp.asarray([p[0] for p in pairs], dtype=jnp.int32)
    bj_arr = jnp.asarray([p[1] for p in pairs], dtype=jnp.int32)

    d = w.shape[1]
    grid_spec = pltpu.PrefetchScalarGridSpec(
        num_scalar_prefetch=2,
        grid=(num_pairs,),
        in_specs=[
            pl.BlockSpec((m, d), lambda t, bi, bj: (0, 0)),
        ],
        out_specs=[
            pl.BlockSpec((t_blocks, _B, 128), lambda t, bi, bj: (0, 0, 0)),
            pl.BlockSpec((t_blocks, 8, _B), lambda t, bi, bj: (0, 0, 0)),
        ],
        scratch_shapes=[],
    )
    outr, outc = pl.pallas_call(
        functools.partial(_body, num_pairs=num_pairs),
        grid_spec=grid_spec,
        out_shape=[
            jax.ShapeDtypeStruct((t_blocks, _B, 128), jnp.float32),
            jax.ShapeDtypeStruct((t_blocks, 8, _B), jnp.float32),
        ],
        compiler_params=pltpu.CompilerParams(
            dimension_semantics=("arbitrary",),
        ),
    )(bi_arr, bj_arr, w)
    return jnp.sum(outr, axis=2).reshape(m) + jnp.sum(outc, axis=1).reshape(m)


def kernel(h1, h2):
    n, d = h1.shape
    z1 = h1 / jnp.linalg.norm(h1, axis=1, keepdims=True)
    z2 = h2 / jnp.linalg.norm(h2, axis=1, keepdims=True)

    inv_tau = jnp.float32(1.0 / _TAU)
    c = float(1.0 / _TAU) * 1.4426950408889634
    sqrt_c = c ** 0.5

    np_ = ((n + _B - 1) // _B) * _B
    pad = np_ - n
    z1p = jnp.pad(z1 * sqrt_c, ((0, pad), (0, 0))).astype(jnp.bfloat16)
    z2p = jnp.pad(z2 * sqrt_c, ((0, pad), (0, 0))).astype(jnp.bfloat16)
    w = jnp.concatenate([z1p, z2p], axis=0)
    m = 2 * np_

    g = _rowsums(w, m)
    s1 = g[:n]
    s2 = g[np_ : np_ + n]

    self_sim = jnp.exp(inv_tau)
    pad_ones = jnp.float32(2 * pad)
    denom1 = s1 - pad_ones - self_sim
    denom2 = s2 - pad_ones - self_sim
    log_pos = jnp.sum(z1 * z2, axis=1) * inv_tau
    l1 = jnp.mean(jnp.log(denom1) - log_pos)
    l2 = jnp.mean(jnp.log(denom2) - log_pos)
    return (l1 + l2) * jnp.float32(0.5)
